# baseline (speedup 1.0000x reference)
import functools

import jax
import jax.numpy as jnp
from jax import lax
from jax.experimental import pallas as pl
from jax.experimental.pallas import tpu as pltpu
from jax.experimental.pallas import tpu_sc as plsc

_B = 16384
_D = 64
_NC = 2
_NS = 16
_NW = _NC * _NS
_BPW = _B // _NW
_CHUNK = 128
_L = 16
_CB = 2048

_MESH = plsc.VectorSubcoreMesh(
    core_axis_name="c", subcore_axis_name="s",
    num_cores=_NC, num_subcores=_NS)


def _pack_pairs_body(x_ref, o_ref):
    t = jnp.transpose(x_ref[...], (1, 0))
    t3 = t.reshape(_CB // 2, 2, _D)
    o_ref[:, 0:_D] = t3[:, 0, :]
    o_ref[:, _D:2 * _D] = t3[:, 1, :]


def _pack_pairs(table_t):
    n = table_t.shape[1]
    return pl.pallas_call(
        _pack_pairs_body,
        grid=((n + _CB - 1) // _CB,),
        in_specs=[pl.BlockSpec((_D, _CB), lambda t: (0, t))],
        out_specs=pl.BlockSpec((_CB // 2, 2 * _D), lambda t: (t, 0)),
        out_shape=jax.ShapeDtypeStruct((n // 2, 2 * _D), jnp.float32),
    )(table_t)


def _gather_kernel(n_lookups):
    row = jax.ShapeDtypeStruct((_B, 2 * _D), jnp.float32)

    @functools.partial(
        pl.kernel,
        mesh=_MESH,
        out_type=(row,) * n_lookups,
        compiler_params=pltpu.CompilerParams(use_tc_tiling_on_sc=True,
                                             needs_layout_passes=False),
        scratch_types=[
            *[pltpu.VMEM((_BPW,), jnp.int32) for _ in range(n_lookups)],
            *[pltpu.VMEM((_BPW // 2, 2 * _D), jnp.float32)
              for _ in range(n_lookups)],
            *[pltpu.SemaphoreType.DMA for _ in range(n_lookups)],
            pltpu.SemaphoreType.DMA,
        ],
    )
    def body(tbl, *rest):
        idx_hs = rest[:n_lookups]
        outs = rest[n_lookups:2 * n_lookups]
        pidxs = rest[2 * n_lookups:3 * n_lookups]
        rowss = rest[3 * n_lookups:4 * n_lookups]
        gsems = rest[4 * n_lookups:5 * n_lookups]
        wsem = rest[5 * n_lookups]
        wid = lax.axis_index("s") * _NC + lax.axis_index("c")
        base = wid * _BPW
        half = _BPW // 2
        for idx_h, pidx in zip(idx_hs, pidxs):
            pltpu.sync_copy(idx_h.at[pl.ds(base, _BPW)], pidx)
            for c in range(_BPW // _L):
                sl = pl.ds(c * _L, _L)
                pidx[sl] = lax.shift_right_logical(pidx[sl], 1)
        writes = []
        for hh in range(2):
            gathers = [[] for _ in range(n_lookups)]
            for c in range(half // _CHUNK):
                src_sl = pl.ds(hh * half + c * _CHUNK, _CHUNK)
                dst_sl = pl.ds(c * _CHUNK, _CHUNK)
                for t in range(n_lookups):
                    gathers[t].append(
                        pltpu.async_copy(tbl.at[pidxs[t].at[src_sl]],
                                         rowss[t].at[dst_sl, :], gsems[t]))
            for t in range(n_lookups):
                for h in gathers[t]:
                    h.wait()
                writes.append(
                    pltpu.async_copy(rowss[t],
                                     outs[t].at[pl.ds(base + hh * half, half)],
                                     wsem))
            if hh == 0:
                for h in writes:
                    h.wait()
                writes = []
        for h in writes:
            h.wait()

    return body


def _take_half(pair_rows, idx):
    odd = (idx & 1)[:, None] == 1
    return jnp.where(odd, pair_rows[:, _D:2 * _D], pair_rows[:, 0:_D])


def kernel(user, pos_item, neg_item, user_emb, item_emb):
    up = _pack_pairs(user_emb.T)
    ip = _pack_pairs(item_emb.T)
    (u2,) = _gather_kernel(1)(up, user)
    i2, j2 = _gather_kernel(2)(ip, pos_item, neg_item)
    return (_take_half(u2, user), _take_half(i2, pos_item),
            _take_half(j2, neg_item))


# speedup vs baseline: 1.1971x; 1.1971x over previous
import functools

import jax
import jax.numpy as jnp
from jax import lax
from jax.experimental import pallas as pl
from jax.experimental.pallas import tpu as pltpu
from jax.experimental.pallas import tpu_sc as plsc

_B = 16384
_D = 64
_NC = 2
_NS = 16
_NW = _NC * _NS
_BPW = _B // _NW
_CHUNK = 128
_L = 16
_CB = 2048

_MESH = plsc.VectorSubcoreMesh(
    core_axis_name="c", subcore_axis_name="s",
    num_cores=_NC, num_subcores=_NS)


def _pack_pairs_body(x_ref, o_ref):
    t = jnp.transpose(x_ref[...], (1, 0))
    o_ref[:, 0:_D] = t[0:_CB // 2]
    o_ref[:, _D:2 * _D] = t[_CB // 2:_CB]


def _pack_pairs(table_t):
    n = table_t.shape[1]
    nblk = (n + _CB - 1) // _CB
    return pl.pallas_call(
        _pack_pairs_body,
        grid=(nblk,),
        in_specs=[pl.BlockSpec((_D, _CB), lambda t: (0, t))],
        out_specs=pl.BlockSpec((_CB // 2, 2 * _D), lambda t: (t, 0)),
        out_shape=jax.ShapeDtypeStruct((nblk * (_CB // 2), 2 * _D),
                                       jnp.float32),
    )(table_t)


def _gather_kernel(n_lookups):
    row = jax.ShapeDtypeStruct((_B, 2 * _D), jnp.float32)

    @functools.partial(
        pl.kernel,
        mesh=_MESH,
        out_type=(row,) * n_lookups,
        compiler_params=pltpu.CompilerParams(use_tc_tiling_on_sc=True,
                                             needs_layout_passes=False),
        scratch_types=[
            *[pltpu.VMEM((_BPW,), jnp.int32) for _ in range(n_lookups)],
            *[pltpu.VMEM((_BPW // 2, 2 * _D), jnp.float32)
              for _ in range(n_lookups)],
            *[pltpu.SemaphoreType.DMA for _ in range(n_lookups)],
            pltpu.SemaphoreType.DMA,
        ],
    )
    def body(tbl, *rest):
        idx_hs = rest[:n_lookups]
        outs = rest[n_lookups:2 * n_lookups]
        pidxs = rest[2 * n_lookups:3 * n_lookups]
        rowss = rest[3 * n_lookups:4 * n_lookups]
        gsems = rest[4 * n_lookups:5 * n_lookups]
        wsem = rest[5 * n_lookups]
        wid = lax.axis_index("s") * _NC + lax.axis_index("c")
        base = wid * _BPW
        half = _BPW // 2
        for idx_h, pidx in zip(idx_hs, pidxs):
            pltpu.sync_copy(idx_h.at[pl.ds(base, _BPW)], pidx)
            for c in range(_BPW // _L):
                sl = pl.ds(c * _L, _L)
                v = pidx[sl]
                pidx[sl] = (
                    lax.shift_left(lax.shift_right_logical(v, 11), 10)
                    + (v & (_CB // 2 - 1)))
        writes = []
        for hh in range(2):
            gathers = [[] for _ in range(n_lookups)]
            for c in range(half // _CHUNK):
                src_sl = pl.ds(hh * half + c * _CHUNK, _CHUNK)
                dst_sl = pl.ds(c * _CHUNK, _CHUNK)
                for t in range(n_lookups):
                    gathers[t].append(
                        pltpu.async_copy(tbl.at[pidxs[t].at[src_sl]],
                                         rowss[t].at[dst_sl, :], gsems[t]))
            for t in range(n_lookups):
                for h in gathers[t]:
                    h.wait()
                writes.append(
                    pltpu.async_copy(rowss[t],
                                     outs[t].at[pl.ds(base + hh * half, half)],
                                     wsem))
            if hh == 0:
                for h in writes:
                    h.wait()
                writes = []
        for h in writes:
            h.wait()

    return body


def _take_half(pair_rows, idx):
    hi = (lax.shift_right_logical(idx, 10) & 1)[:, None] == 1
    return jnp.where(hi, pair_rows[:, _D:2 * _D], pair_rows[:, 0:_D])


def kernel(user, pos_item, neg_item, user_emb, item_emb):
    up = _pack_pairs(user_emb.T)
    ip = _pack_pairs(item_emb.T)
    (u2,) = _gather_kernel(1)(up, user)
    i2, j2 = _gather_kernel(2)(ip, pos_item, neg_item)
    return (_take_half(u2, user), _take_half(i2, pos_item),
            _take_half(j2, neg_item))


# speedup vs baseline: 1.5942x; 1.3317x over previous
import functools

import jax
import jax.numpy as jnp
from jax import lax
from jax.experimental import pallas as pl
from jax.experimental.pallas import tpu as pltpu
from jax.experimental.pallas import tpu_sc as plsc

_B = 16384
_D = 64
_NC = 2
_NS = 16
_NW = _NC * _NS
_BPW = _B // _NW
_CHUNK = 128
_L = 16
_CB = 4096
_CBL = _CB.bit_length() - 1

_MESH = plsc.VectorSubcoreMesh(
    core_axis_name="c", subcore_axis_name="s",
    num_cores=_NC, num_subcores=_NS)


def _pack_pairs_body(x_ref, o_ref):
    t = jnp.transpose(x_ref[...], (1, 0))
    o_ref[:, 0:_D] = t[0:_CB // 2]
    o_ref[:, _D:2 * _D] = t[_CB // 2:_CB]


def _pack_pairs(table_t):
    n = table_t.shape[1]
    nblk = (n + _CB - 1) // _CB
    return pl.pallas_call(
        _pack_pairs_body,
        grid=(nblk,),
        in_specs=[pl.BlockSpec((_D, _CB), lambda t: (0, t))],
        out_specs=pl.BlockSpec((_CB // 2, 2 * _D), lambda t: (t, 0)),
        out_shape=jax.ShapeDtypeStruct((nblk * (_CB // 2), 2 * _D),
                                       jnp.float32),
    )(table_t)


def _gather_kernel(n_lookups):
    row = jax.ShapeDtypeStruct((_B, 2 * _D), jnp.float32)

    @functools.partial(
        pl.kernel,
        mesh=_MESH,
        out_type=(row,) * n_lookups,
        compiler_params=pltpu.CompilerParams(use_tc_tiling_on_sc=True,
                                             needs_layout_passes=False),
        scratch_types=[
            *[pltpu.VMEM((_BPW,), jnp.int32) for _ in range(n_lookups)],
            *[pltpu.VMEM((_BPW // 2, 2 * _D), jnp.float32)
              for _ in range(n_lookups)],
            *[pltpu.SemaphoreType.DMA for _ in range(n_lookups)],
            pltpu.SemaphoreType.DMA,
        ],
    )
    def body(tbl, *rest):
        idx_hs = rest[:n_lookups]
        outs = rest[n_lookups:2 * n_lookups]
        pidxs = rest[2 * n_lookups:3 * n_lookups]
        rowss = rest[3 * n_lookups:4 * n_lookups]
        gsems = rest[4 * n_lookups:5 * n_lookups]
        wsem = rest[5 * n_lookups]
        wid = lax.axis_index("s") * _NC + lax.axis_index("c")
        base = wid * _BPW
        half = _BPW // 2
        for idx_h, pidx in zip(idx_hs, pidxs):
            pltpu.sync_copy(idx_h.at[pl.ds(base, _BPW)], pidx)
            for c in range(_BPW // _L):
                sl = pl.ds(c * _L, _L)
                v = pidx[sl]
                pidx[sl] = (
                    lax.shift_left(lax.shift_right_logical(v, _CBL),
                                   _CBL - 1)
                    + (v & (_CB // 2 - 1)))
        writes = []
        for hh in range(2):
            gathers = [[] for _ in range(n_lookups)]
            for c in range(half // _CHUNK):
                src_sl = pl.ds(hh * half + c * _CHUNK, _CHUNK)
                dst_sl = pl.ds(c * _CHUNK, _CHUNK)
                for t in range(n_lookups):
                    gathers[t].append(
                        pltpu.async_copy(tbl.at[pidxs[t].at[src_sl]],
                                         rowss[t].at[dst_sl, :], gsems[t]))
            for t in range(n_lookups):
                for h in gathers[t]:
                    h.wait()
                writes.append(
                    pltpu.async_copy(rowss[t],
                                     outs[t].at[pl.ds(base + hh * half, half)],
                                     wsem))
            if hh == 0:
                for h in writes:
                    h.wait()
                writes = []
        for h in writes:
            h.wait()

    return body


def _take_half(pair_rows, idx):
    hi = (lax.shift_right_logical(idx, _CBL - 1) & 1)[:, None] == 1
    return jnp.where(hi, pair_rows[:, _D:2 * _D], pair_rows[:, 0:_D])


def kernel(user, pos_item, neg_item, user_emb, item_emb):
    up = _pack_pairs(user_emb.T)
    ip = _pack_pairs(item_emb.T)
    (u2,) = _gather_kernel(1)(up, user)
    i2, j2 = _gather_kernel(2)(ip, pos_item, neg_item)
    return (_take_half(u2, user), _take_half(i2, pos_item),
            _take_half(j2, neg_item))


# speedup vs baseline: 1.9514x; 1.2241x over previous
import functools

import jax
import jax.numpy as jnp
from jax import lax
from jax.experimental import pallas as pl
from jax.experimental.pallas import tpu as pltpu
from jax.experimental.pallas import tpu_sc as plsc

_B = 16384
_D = 64
_NC = 2
_NS = 16
_NW = _NC * _NS
_BPW = _B // _NW
_CHUNK = 128
_L = 16
_CB = 8192
_CBL = _CB.bit_length() - 1

_MESH = plsc.VectorSubcoreMesh(
    core_axis_name="c", subcore_axis_name="s",
    num_cores=_NC, num_subcores=_NS)


def _pack_pairs_body(x_ref, o_ref):
    t = jnp.transpose(x_ref[...], (1, 0))
    o_ref[:, 0:_D] = t[0:_CB // 2]
    o_ref[:, _D:2 * _D] = t[_CB // 2:_CB]


def _pack_pairs(table_t):
    n = table_t.shape[1]
    nblk = (n + _CB - 1) // _CB
    return pl.pallas_call(
        _pack_pairs_body,
        grid=(nblk,),
        in_specs=[pl.BlockSpec((_D, _CB), lambda t: (0, t))],
        out_specs=pl.BlockSpec((_CB // 2, 2 * _D), lambda t: (t, 0)),
        out_shape=jax.ShapeDtypeStruct((nblk * (_CB // 2), 2 * _D),
                                       jnp.float32),
    )(table_t)


def _gather_kernel(n_lookups):
    row = jax.ShapeDtypeStruct((_B, 2 * _D), jnp.float32)

    @functools.partial(
        pl.kernel,
        mesh=_MESH,
        out_type=(row,) * n_lookups,
        compiler_params=pltpu.CompilerParams(use_tc_tiling_on_sc=True,
                                             needs_layout_passes=False),
        scratch_types=[
            *[pltpu.VMEM((_BPW,), jnp.int32) for _ in range(n_lookups)],
            *[pltpu.VMEM((_BPW // 2, 2 * _D), jnp.float32)
              for _ in range(n_lookups)],
            *[pltpu.SemaphoreType.DMA for _ in range(n_lookups)],
            pltpu.SemaphoreType.DMA,
        ],
    )
    def body(tbl, *rest):
        idx_hs = rest[:n_lookups]
        outs = rest[n_lookups:2 * n_lookups]
        pidxs = rest[2 * n_lookups:3 * n_lookups]
        rowss = rest[3 * n_lookups:4 * n_lookups]
        gsems = rest[4 * n_lookups:5 * n_lookups]
        wsem = rest[5 * n_lookups]
        wid = lax.axis_index("s") * _NC + lax.axis_index("c")
        base = wid * _BPW
        half = _BPW // 2
        for idx_h, pidx in zip(idx_hs, pidxs):
            pltpu.sync_copy(idx_h.at[pl.ds(base, _BPW)], pidx)
            for c in range(_BPW // _L):
                sl = pl.ds(c * _L, _L)
                v = pidx[sl]
                pidx[sl] = (
                    lax.shift_left(lax.shift_right_logical(v, _CBL),
                                   _CBL - 1)
                    + (v & (_CB // 2 - 1)))
        writes = []
        for hh in range(2):
            gathers = [[] for _ in range(n_lookups)]
            for c in range(half // _CHUNK):
                src_sl = pl.ds(hh * half + c * _CHUNK, _CHUNK)
                dst_sl = pl.ds(c * _CHUNK, _CHUNK)
                for t in range(n_lookups):
                    gathers[t].append(
                        pltpu.async_copy(tbl.at[pidxs[t].at[src_sl]],
                                         rowss[t].at[dst_sl, :], gsems[t]))
            for t in range(n_lookups):
                for h in gathers[t]:
                    h.wait()
                writes.append(
                    pltpu.async_copy(rowss[t],
                                     outs[t].at[pl.ds(base + hh * half, half)],
                                     wsem))
            if hh == 0:
                for h in writes:
                    h.wait()
                writes = []
        for h in writes:
            h.wait()

    return body


def _take_half(pair_rows, idx):
    hi = (lax.shift_right_logical(idx, _CBL - 1) & 1)[:, None] == 1
    return jnp.where(hi, pair_rows[:, _D:2 * _D], pair_rows[:, 0:_D])


def kernel(user, pos_item, neg_item, user_emb, item_emb):
    up = _pack_pairs(user_emb.T)
    ip = _pack_pairs(item_emb.T)
    (u2,) = _gather_kernel(1)(up, user)
    i2, j2 = _gather_kernel(2)(ip, pos_item, neg_item)
    return (_take_half(u2, user), _take_half(i2, pos_item),
            _take_half(j2, neg_item))


# speedup vs baseline: 2.1995x; 1.1272x over previous
import functools

import jax
import jax.numpy as jnp
from jax import lax
from jax.experimental import pallas as pl
from jax.experimental.pallas import tpu as pltpu
from jax.experimental.pallas import tpu_sc as plsc

_B = 16384
_D = 64
_NC = 2
_NS = 16
_NW = _NC * _NS
_BPW = _B // _NW
_CHUNK = 128
_L = 16
_CB = 16384
_CBL = _CB.bit_length() - 1

_MESH = plsc.VectorSubcoreMesh(
    core_axis_name="c", subcore_axis_name="s",
    num_cores=_NC, num_subcores=_NS)


def _pack_pairs_body(x_ref, o_ref):
    t = jnp.transpose(x_ref[...], (1, 0))
    o_ref[:, 0:_D] = t[0:_CB // 2]
    o_ref[:, _D:2 * _D] = t[_CB // 2:_CB]


def _pack_pairs(table_t):
    n = table_t.shape[1]
    nblk = (n + _CB - 1) // _CB
    return pl.pallas_call(
        _pack_pairs_body,
        grid=(nblk,),
        in_specs=[pl.BlockSpec((_D, _CB), lambda t: (0, t))],
        out_specs=pl.BlockSpec((_CB // 2, 2 * _D), lambda t: (t, 0)),
        out_shape=jax.ShapeDtypeStruct((nblk * (_CB // 2), 2 * _D),
                                       jnp.float32),
    )(table_t)


def _gather_kernel(n_lookups):
    row = jax.ShapeDtypeStruct((_B, 2 * _D), jnp.float32)

    @functools.partial(
        pl.kernel,
        mesh=_MESH,
        out_type=(row,) * n_lookups,
        compiler_params=pltpu.CompilerParams(use_tc_tiling_on_sc=True,
                                             needs_layout_passes=False),
        scratch_types=[
            *[pltpu.VMEM((_BPW,), jnp.int32) for _ in range(n_lookups)],
            *[pltpu.VMEM((_BPW // 2, 2 * _D), jnp.float32)
              for _ in range(n_lookups)],
            *[pltpu.SemaphoreType.DMA for _ in range(n_lookups)],
            pltpu.SemaphoreType.DMA,
        ],
    )
    def body(tbl, *rest):
        idx_hs = rest[:n_lookups]
        outs = rest[n_lookups:2 * n_lookups]
        pidxs = rest[2 * n_lookups:3 * n_lookups]
        rowss = rest[3 * n_lookups:4 * n_lookups]
        gsems = rest[4 * n_lookups:5 * n_lookups]
        wsem = rest[5 * n_lookups]
        wid = lax.axis_index("s") * _NC + lax.axis_index("c")
        base = wid * _BPW
        half = _BPW // 2
        for idx_h, pidx in zip(idx_hs, pidxs):
            pltpu.sync_copy(idx_h.at[pl.ds(base, _BPW)], pidx)
            for c in range(_BPW // _L):
                sl = pl.ds(c * _L, _L)
                v = pidx[sl]
                pidx[sl] = (
                    lax.shift_left(lax.shift_right_logical(v, _CBL),
                                   _CBL - 1)
                    + (v & (_CB // 2 - 1)))
        writes = []
        for hh in range(2):
            gathers = [[] for _ in range(n_lookups)]
            for c in range(half // _CHUNK):
                src_sl = pl.ds(hh * half + c * _CHUNK, _CHUNK)
                dst_sl = pl.ds(c * _CHUNK, _CHUNK)
                for t in range(n_lookups):
                    gathers[t].append(
                        pltpu.async_copy(tbl.at[pidxs[t].at[src_sl]],
                                         rowss[t].at[dst_sl, :], gsems[t]))
            for t in range(n_lookups):
                for h in gathers[t]:
                    h.wait()
                writes.append(
                    pltpu.async_copy(rowss[t],
                                     outs[t].at[pl.ds(base + hh * half, half)],
                                     wsem))
            if hh == 0:
                for h in writes:
                    h.wait()
                writes = []
        for h in writes:
            h.wait()

    return body


def _take_half(pair_rows, idx):
    hi = (lax.shift_right_logical(idx, _CBL - 1) & 1)[:, None] == 1
    return jnp.where(hi, pair_rows[:, _D:2 * _D], pair_rows[:, 0:_D])


def kernel(user, pos_item, neg_item, user_emb, item_emb):
    up = _pack_pairs(user_emb.T)
    ip = _pack_pairs(item_emb.T)
    (u2,) = _gather_kernel(1)(up, user)
    i2, j2 = _gather_kernel(2)(ip, pos_item, neg_item)
    return (_take_half(u2, user), _take_half(i2, pos_item),
            _take_half(j2, neg_item))


# speedup vs baseline: 2.3311x; 1.0598x over previous
import functools

import jax
import jax.numpy as jnp
from jax import lax
from jax.experimental import pallas as pl
from jax.experimental.pallas import tpu as pltpu
from jax.experimental.pallas import tpu_sc as plsc

_B = 16384
_D = 64
_NC = 2
_NS = 16
_NW = _NC * _NS
_BPW = _B // _NW
_CHUNK = 128
_L = 16
_CB = 32768
_CBL = _CB.bit_length() - 1

_MESH = plsc.VectorSubcoreMesh(
    core_axis_name="c", subcore_axis_name="s",
    num_cores=_NC, num_subcores=_NS)


def _pack_pairs_body(x_ref, o_ref):
    t = jnp.transpose(x_ref[...], (1, 0))
    o_ref[:, 0:_D] = t[0:_CB // 2]
    o_ref[:, _D:2 * _D] = t[_CB // 2:_CB]


def _pack_pairs(table_t):
    n = table_t.shape[1]
    nblk = (n + _CB - 1) // _CB
    return pl.pallas_call(
        _pack_pairs_body,
        grid=(nblk,),
        in_specs=[pl.BlockSpec((_D, _CB), lambda t: (0, t))],
        out_specs=pl.BlockSpec((_CB // 2, 2 * _D), lambda t: (t, 0)),
        out_shape=jax.ShapeDtypeStruct((nblk * (_CB // 2), 2 * _D),
                                       jnp.float32),
    )(table_t)


def _gather_kernel(n_lookups):
    row = jax.ShapeDtypeStruct((_B, 2 * _D), jnp.float32)

    @functools.partial(
        pl.kernel,
        mesh=_MESH,
        out_type=(row,) * n_lookups,
        compiler_params=pltpu.CompilerParams(use_tc_tiling_on_sc=True,
                                             needs_layout_passes=False),
        scratch_types=[
            *[pltpu.VMEM((_BPW,), jnp.int32) for _ in range(n_lookups)],
            *[pltpu.VMEM((_BPW // 2, 2 * _D), jnp.float32)
              for _ in range(n_lookups)],
            *[pltpu.SemaphoreType.DMA for _ in range(n_lookups)],
            pltpu.SemaphoreType.DMA,
        ],
    )
    def body(tbl, *rest):
        idx_hs = rest[:n_lookups]
        outs = rest[n_lookups:2 * n_lookups]
        pidxs = rest[2 * n_lookups:3 * n_lookups]
        rowss = rest[3 * n_lookups:4 * n_lookups]
        gsems = rest[4 * n_lookups:5 * n_lookups]
        wsem = rest[5 * n_lookups]
        wid = lax.axis_index("s") * _NC + lax.axis_index("c")
        base = wid * _BPW
        half = _BPW // 2
        for idx_h, pidx in zip(idx_hs, pidxs):
            pltpu.sync_copy(idx_h.at[pl.ds(base, _BPW)], pidx)
            for c in range(_BPW // _L):
                sl = pl.ds(c * _L, _L)
                v = pidx[sl]
                pidx[sl] = (
                    lax.shift_left(lax.shift_right_logical(v, _CBL),
                                   _CBL - 1)
                    + (v & (_CB // 2 - 1)))
        writes = []
        for hh in range(2):
            gathers = [[] for _ in range(n_lookups)]
            for c in range(half // _CHUNK):
                src_sl = pl.ds(hh * half + c * _CHUNK, _CHUNK)
                dst_sl = pl.ds(c * _CHUNK, _CHUNK)
                for t in range(n_lookups):
                    gathers[t].append(
                        pltpu.async_copy(tbl.at[pidxs[t].at[src_sl]],
                                         rowss[t].at[dst_sl, :], gsems[t]))
            for t in range(n_lookups):
                for h in gathers[t]:
                    h.wait()
                writes.append(
                    pltpu.async_copy(rowss[t],
                                     outs[t].at[pl.ds(base + hh * half, half)],
                                     wsem))
            if hh == 0:
                for h in writes:
                    h.wait()
                writes = []
        for h in writes:
            h.wait()

    return body


def _take_half(pair_rows, idx):
    hi = (lax.shift_right_logical(idx, _CBL - 1) & 1)[:, None] == 1
    return jnp.where(hi, pair_rows[:, _D:2 * _D], pair_rows[:, 0:_D])


def kernel(user, pos_item, neg_item, user_emb, item_emb):
    up = _pack_pairs(user_emb.T)
    ip = _pack_pairs(item_emb.T)
    (u2,) = _gather_kernel(1)(up, user)
    i2, j2 = _gather_kernel(2)(ip, pos_item, neg_item)
    return (_take_half(u2, user), _take_half(i2, pos_item),
            _take_half(j2, neg_item))
